# baseline (speedup 1.0000x reference)
import functools

import jax
import jax.numpy as jnp
from jax import lax
from jax.experimental import pallas as pl
from jax.experimental.pallas import tpu as pltpu
from jax.experimental.pallas import tpu_sc as plsc

M = 204800
B = 4096
H = 256
SV = 65536
NC = 2
NS = 16
NW = NC * NS
C = 64
PER_W = M // NW
CHUNKS_PER_W = PER_W // C
N_CHUNKS = M // C

TK_BLK = 2048
TK_GRID = SV // TK_BLK


def _prep(img_z, role_emb, str_emb, t_emb, k_emb, face_emb, w1i, b1):
    def body(img_ref, role_ref, str_ref, t_ref, k_ref, f_ref, w1_ref, b1_ref,
             a_img, str128, rf, tk):
        i = pl.program_id(0)
        z64 = jnp.zeros((TK_BLK, 64), jnp.float32)
        str128[...] = jnp.concatenate([str_ref[...], z64], axis=1)

        t_blk = t_ref[...]
        t_rep = jnp.broadcast_to(t_blk[:, None, :], (8, 256, 16))
        t_rep = t_rep.reshape(TK_BLK, 16)
        k_rep = jnp.broadcast_to(k_ref[...][None, :, :], (8, 256, 16))
        k_rep = k_rep.reshape(TK_BLK, 16)
        tk[...] = jnp.concatenate(
            [t_rep, k_rep, jnp.zeros((TK_BLK, 96), jnp.float32)], axis=1)

        @pl.when(i == 0)
        def _():
            a_img[...] = (jnp.dot(img_ref[...], w1_ref[...],
                                  preferred_element_type=jnp.float32,
                                  precision=lax.Precision.HIGHEST)
                          + b1_ref[...])
            role_rep = jnp.broadcast_to(role_ref[...][:, None, :], (8, 8, 16))
            role_rep = role_rep.reshape(64, 16)
            f_pad = jnp.concatenate(
                [f_ref[...], jnp.zeros((8, 8), jnp.float32)], axis=1)
            f_rep = jnp.broadcast_to(f_pad[None, :, :], (8, 8, 16))
            f_rep = f_rep.reshape(64, 16)
            rf[...] = jnp.concatenate(
                [role_rep, f_rep, jnp.zeros((64, 96), jnp.float32)], axis=1)

    full = lambda shape: pl.BlockSpec(shape, lambda i: tuple(0 for _ in shape))
    return pl.pallas_call(
        body,
        grid=(TK_GRID,),
        in_specs=[
            full((B, 256)),
            full((8, 16)),
            pl.BlockSpec((TK_BLK, 64), lambda i: (i, 0)),
            pl.BlockSpec((8, 16), lambda i: (i, 0)),
            full((256, 16)),
            full((8, 8)),
            full((256, H)),
            full((1, H)),
        ],
        out_specs=[
            full((B, H)),
            pl.BlockSpec((TK_BLK, 128), lambda i: (i, 0)),
            full((64, 128)),
            pl.BlockSpec((TK_BLK, 128), lambda i: (i, 0)),
        ],
        out_shape=[
            jax.ShapeDtypeStruct((B, H), jnp.float32),
            jax.ShapeDtypeStruct((SV, 128), jnp.float32),
            jax.ShapeDtypeStruct((64, 128), jnp.float32),
            jax.ShapeDtypeStruct((SV, 128), jnp.float32),
        ],
    )(img_z, role_emb, str_emb, t_emb, k_emb, face_emb, w1i, b1)


def _sc_gather(idxs, a_img, rf_t, str_t, tk_t):
    mesh = plsc.VectorSubcoreMesh(core_axis_name="c", subcore_axis_name="s")

    fields = [(1, 256, 32), (2, 288, 64), (3, 352, 64), (4, 416, 64),
              (5, 480, 64), (6, 544, 32)]

    @functools.partial(
        pl.kernel,
        out_type=jax.ShapeDtypeStruct((M, 576), jnp.float32),
        mesh=mesh,
        scratch_types=[
            pltpu.VMEM((8, C), jnp.int32),
            pltpu.VMEM((C, 576), jnp.float32),
            pltpu.VMEM((C, 128), jnp.float32),
            pltpu.VMEM((C, 128), jnp.float32),
            pltpu.VMEM((C, 128), jnp.float32),
            pltpu.VMEM((C, 128), jnp.float32),
            pltpu.VMEM((C, 128), jnp.float32),
            pltpu.VMEM((C, 128), jnp.float32),
            pltpu.SemaphoreType.DMA,
        ],
    )
    def k(idxs_hbm, img_hbm, rf_hbm, str_hbm, tk_hbm, g_out,
          ibuf, pbuf, brf, bpred, bop, bnt, bpu, btk, gsem):
        wid = lax.axis_index("s") * NC + lax.axis_index("c")
        bufs = [brf, bpred, bop, bnt, bpu, btk]

        def chunk(c, carry):
            g = wid * CHUNKS_PER_W + c
            pltpu.sync_copy(idxs_hbm.at[g], ibuf)
            cps = [
                pltpu.async_copy(img_hbm.at[ibuf.at[0]],
                                 pbuf.at[:, pl.ds(0, 256)], gsem),
                pltpu.async_copy(rf_hbm.at[ibuf.at[1]], brf, gsem),
                pltpu.async_copy(str_hbm.at[ibuf.at[2]], bpred, gsem),
                pltpu.async_copy(str_hbm.at[ibuf.at[3]], bop, gsem),
                pltpu.async_copy(str_hbm.at[ibuf.at[4]], bnt, gsem),
                pltpu.async_copy(str_hbm.at[ibuf.at[5]], bpu, gsem),
                pltpu.async_copy(tk_hbm.at[ibuf.at[6]], btk, gsem),
            ]
            for cp in cps:
                cp.wait()

            def pack_row(r, carry2):
                for buf, (_, off, w) in zip(bufs, fields):
                    for j in range(w // 16):
                        pbuf[r, pl.ds(off + j * 16, 16)] = (
                            buf[r, pl.ds(j * 16, 16)])
                return carry2

            lax.fori_loop(0, C, pack_row, 0)
            pltpu.sync_copy(pbuf, g_out.at[pl.ds(g * C, C), :])
            return carry

        lax.fori_loop(0, CHUNKS_PER_W, chunk, 0)

    return k(idxs, a_img, rf_t, str_t, tk_t)


def _head(g_feat, w1rest, w2, b2):
    bm = 2048

    def body(g_ref, w1_ref, w2_ref, b2_ref, out_ref):
        x = g_ref[...]
        h = x[:, :256] + jnp.dot(x[:, 256:], w1_ref[...],
                                 preferred_element_type=jnp.float32,
                                 precision=lax.Precision.HIGHEST)
        h = jnp.maximum(h, 0.0)
        out_ref[...] = (jnp.sum(h * w2_ref[...], axis=1, keepdims=True)
                        + b2_ref[...])

    out = pl.pallas_call(
        body,
        grid=(M // bm,),
        in_specs=[
            pl.BlockSpec((bm, 576), lambda i: (i, 0)),
            pl.BlockSpec((320, H), lambda i: (0, 0)),
            pl.BlockSpec((1, H), lambda i: (0, 0)),
            pl.BlockSpec((1, 1), lambda i: (0, 0)),
        ],
        out_specs=pl.BlockSpec((bm, 1), lambda i: (i, 0)),
        out_shape=jax.ShapeDtypeStruct((M, 1), jnp.float32),
    )(g_feat, w1rest, w2, b2)
    return out[:, 0]


def kernel(img_z, desc_batch_idx, role_idx, pred_i, op_i, nt_i, pu_i,
           t_idx, k_idx, f_idx, role_emb, str_emb, t_emb, k_emb, face_emb,
           W1, b1, W2, b2):
    i32 = jnp.int32
    rf_i = role_idx.astype(i32) * 8 + f_idx.astype(i32)
    tk_i = t_idx.astype(i32) * 256 + k_idx.astype(i32)
    zeros = jnp.zeros((M,), i32)
    idxs = jnp.stack([
        desc_batch_idx.astype(i32), rf_i, pred_i.astype(i32),
        op_i.astype(i32), nt_i.astype(i32), pu_i.astype(i32), tk_i, zeros,
    ])
    idxs = idxs.reshape(8, N_CHUNKS, C).transpose(1, 0, 2)

    w1t = W1.T
    w1rest = jnp.concatenate([
        w1t[256:272],
        w1t[560:568],
        jnp.zeros((8, H), jnp.float32),
        w1t[272:528],
        w1t[528:560],
    ], axis=0)

    a_img, str_t, rf_t, tk_t = _prep(img_z, role_emb, str_emb, t_emb, k_emb,
                                     face_emb, w1t[0:256], b1.reshape(1, H))
    g = _sc_gather(idxs, a_img, rf_t, str_t, tk_t)
    return _head(g, w1rest, W2.reshape(1, H), b2.reshape(1, 1))


# speedup vs baseline: 1.2761x; 1.2761x over previous
import functools

import jax
import jax.numpy as jnp
from jax import lax
from jax.experimental import pallas as pl
from jax.experimental.pallas import tpu as pltpu
from jax.experimental.pallas import tpu_sc as plsc

M = 204800
B = 4096
H = 256
SV = 65536
NC = 2
NS = 16
NW = NC * NS
C = 40
PER_W = M // NW
CHUNKS_PER_W = PER_W // C
N_CHUNKS = M // C

TK_BLK = 2048
TK_GRID = SV // TK_BLK


def _prep(img_z, role_emb, str_emb, t_emb, k_emb, face_emb, w1i, b1):
    def body(img_ref, role_ref, str_ref, t_ref, k_ref, f_ref, w1_ref, b1_ref,
             a_img, str128, rf, tk):
        i = pl.program_id(0)
        z64 = jnp.zeros((TK_BLK, 64), jnp.float32)
        str128[...] = jnp.concatenate([str_ref[...], z64], axis=1)

        t_blk = t_ref[...]
        t_rep = jnp.broadcast_to(t_blk[:, None, :], (8, 256, 16))
        t_rep = t_rep.reshape(TK_BLK, 16)
        k_rep = jnp.broadcast_to(k_ref[...][None, :, :], (8, 256, 16))
        k_rep = k_rep.reshape(TK_BLK, 16)
        tk[...] = jnp.concatenate(
            [t_rep, k_rep, jnp.zeros((TK_BLK, 96), jnp.float32)], axis=1)

        @pl.when(i == 0)
        def _():
            a_img[...] = (jnp.dot(img_ref[...], w1_ref[...],
                                  preferred_element_type=jnp.float32,
                                  precision=lax.Precision.HIGHEST)
                          + b1_ref[...])
            role_rep = jnp.broadcast_to(role_ref[...][:, None, :], (8, 8, 16))
            role_rep = role_rep.reshape(64, 16)
            f_pad = jnp.concatenate(
                [f_ref[...], jnp.zeros((8, 8), jnp.float32)], axis=1)
            f_rep = jnp.broadcast_to(f_pad[None, :, :], (8, 8, 16))
            f_rep = f_rep.reshape(64, 16)
            rf[...] = jnp.concatenate(
                [role_rep, f_rep, jnp.zeros((64, 96), jnp.float32)], axis=1)

    full = lambda shape: pl.BlockSpec(shape, lambda i: tuple(0 for _ in shape))
    return pl.pallas_call(
        body,
        grid=(TK_GRID,),
        in_specs=[
            full((B, 256)),
            full((8, 16)),
            pl.BlockSpec((TK_BLK, 64), lambda i: (i, 0)),
            pl.BlockSpec((8, 16), lambda i: (i, 0)),
            full((256, 16)),
            full((8, 8)),
            full((256, H)),
            full((1, H)),
        ],
        out_specs=[
            full((B, H)),
            pl.BlockSpec((TK_BLK, 128), lambda i: (i, 0)),
            full((64, 128)),
            pl.BlockSpec((TK_BLK, 128), lambda i: (i, 0)),
        ],
        out_shape=[
            jax.ShapeDtypeStruct((B, H), jnp.float32),
            jax.ShapeDtypeStruct((SV, 128), jnp.float32),
            jax.ShapeDtypeStruct((64, 128), jnp.float32),
            jax.ShapeDtypeStruct((SV, 128), jnp.float32),
        ],
    )(img_z, role_emb, str_emb, t_emb, k_emb, face_emb, w1i, b1)


def _sc_gather(idxs, a_img, rf_t, str_t, tk_t):
    mesh = plsc.VectorSubcoreMesh(core_axis_name="c", subcore_axis_name="s")

    fields = [(1, 256, 32), (2, 288, 64), (3, 352, 64), (4, 416, 64),
              (5, 480, 64), (6, 544, 32)]

    @functools.partial(
        pl.kernel,
        out_type=jax.ShapeDtypeStruct((M, 576), jnp.float32),
        mesh=mesh,
        scratch_types=[
            pltpu.VMEM((2, 8, C), jnp.int32),
            pltpu.VMEM((2, C, 576), jnp.float32),
            pltpu.VMEM((2, C, 128), jnp.float32),
            pltpu.VMEM((2, C, 128), jnp.float32),
            pltpu.VMEM((2, C, 128), jnp.float32),
            pltpu.VMEM((2, C, 128), jnp.float32),
            pltpu.VMEM((2, C, 128), jnp.float32),
            pltpu.VMEM((2, C, 128), jnp.float32),
            pltpu.SemaphoreType.DMA,
            pltpu.SemaphoreType.DMA,
            pltpu.SemaphoreType.DMA,
            pltpu.SemaphoreType.DMA,
        ],
    )
    def k(idxs_hbm, img_hbm, rf_hbm, str_hbm, tk_hbm, g_out,
          ibuf, pbuf, brf, bpred, bop, bnt, bpu, btk,
          gsem0, gsem1, wsem0, wsem1):
        wid = lax.axis_index("s") * NC + lax.axis_index("c")
        base = wid * CHUNKS_PER_W
        gsems = [gsem0, gsem1]
        wsems = [wsem0, wsem1]
        stage = [brf, bpred, bop, bnt, bpu, btk]
        tabs = [rf_hbm, str_hbm, str_hbm, str_hbm, str_hbm, tk_hbm]

        def fire(c, p):
            pltpu.sync_copy(idxs_hbm.at[base + c], ibuf.at[p])
            pltpu.async_copy(img_hbm.at[ibuf.at[p, 0]],
                             pbuf.at[p, :, pl.ds(0, 256)], gsems[p])
            for jj, (buf, tab) in enumerate(zip(stage, tabs)):
                pltpu.async_copy(tab.at[ibuf.at[p, jj + 1]], buf.at[p],
                                 gsems[p])

        def wait_gathers(p):
            pltpu.make_async_copy(img_hbm.at[pl.ds(0, C), :],
                                  pbuf.at[p, :, pl.ds(0, 256)],
                                  gsems[p]).wait()
            for buf, tab in zip(stage, tabs):
                pltpu.make_async_copy(str_hbm.at[pl.ds(0, C), :], buf.at[p],
                                      gsems[p]).wait()

        def pack(p):
            def row(r, carry2):
                for buf, (_, off, w) in zip(stage, fields):
                    for j in range(w // 16):
                        pbuf[p, r, pl.ds(off + j * 16, 16)] = (
                            buf[p, r, pl.ds(j * 16, 16)])
                return carry2
            lax.fori_loop(0, C, row, 0)

        def fire_write(c, p):
            pltpu.async_copy(pbuf.at[p],
                             g_out.at[pl.ds((base + c) * C, C), :], wsems[p])

        def wait_write(p):
            pltpu.make_async_copy(pbuf.at[p], g_out.at[pl.ds(0, C), :],
                                  wsems[p]).wait()

        fire(0, 0)
        fire(1, 1)
        H2 = CHUNKS_PER_W // 2

        def body(i2, carry):
            for p in (0, 1):
                c = i2 * 2 + p
                wait_gathers(p)
                pack(p)
                fire_write(c, p)

                @pl.when(i2 < H2 - 1)
                def _():
                    wait_write(p)
                    fire(c + 2, p)
            return carry

        lax.fori_loop(0, H2, body, 0)
        wait_write(0)
        wait_write(1)

    return k(idxs, a_img, rf_t, str_t, tk_t)


def _head(g_feat, w1rest, w2, b2):
    bm = 2048

    def body(g_ref, w1_ref, w2_ref, b2_ref, out_ref):
        x = g_ref[...]
        h = x[:, :256] + jnp.dot(x[:, 256:], w1_ref[...],
                                 preferred_element_type=jnp.float32,
                                 precision=lax.Precision.HIGHEST)
        h = jnp.maximum(h, 0.0)
        out_ref[...] = (jnp.sum(h * w2_ref[...], axis=1, keepdims=True)
                        + b2_ref[...])

    out = pl.pallas_call(
        body,
        grid=(M // bm,),
        in_specs=[
            pl.BlockSpec((bm, 576), lambda i: (i, 0)),
            pl.BlockSpec((320, H), lambda i: (0, 0)),
            pl.BlockSpec((1, H), lambda i: (0, 0)),
            pl.BlockSpec((1, 1), lambda i: (0, 0)),
        ],
        out_specs=pl.BlockSpec((bm, 1), lambda i: (i, 0)),
        out_shape=jax.ShapeDtypeStruct((M, 1), jnp.float32),
    )(g_feat, w1rest, w2, b2)
    return out[:, 0]


def kernel(img_z, desc_batch_idx, role_idx, pred_i, op_i, nt_i, pu_i,
           t_idx, k_idx, f_idx, role_emb, str_emb, t_emb, k_emb, face_emb,
           W1, b1, W2, b2):
    i32 = jnp.int32
    rf_i = role_idx.astype(i32) * 8 + f_idx.astype(i32)
    tk_i = t_idx.astype(i32) * 256 + k_idx.astype(i32)
    zeros = jnp.zeros((M,), i32)
    idxs = jnp.stack([
        desc_batch_idx.astype(i32), rf_i, pred_i.astype(i32),
        op_i.astype(i32), nt_i.astype(i32), pu_i.astype(i32), tk_i, zeros,
    ])
    idxs = idxs.reshape(8, N_CHUNKS, C).transpose(1, 0, 2)

    w1t = W1.T
    w1rest = jnp.concatenate([
        w1t[256:272],
        w1t[560:568],
        jnp.zeros((8, H), jnp.float32),
        w1t[272:528],
        w1t[528:560],
    ], axis=0)

    a_img, str_t, rf_t, tk_t = _prep(img_z, role_emb, str_emb, t_emb, k_emb,
                                     face_emb, w1t[0:256], b1.reshape(1, H))
    g = _sc_gather(idxs, a_img, rf_t, str_t, tk_t)
    return _head(g, w1rest, W2.reshape(1, H), b2.reshape(1, 1))


# speedup vs baseline: 1.4980x; 1.1739x over previous
import functools

import jax
import jax.numpy as jnp
from jax import lax
from jax.experimental import pallas as pl
from jax.experimental.pallas import tpu as pltpu
from jax.experimental.pallas import tpu_sc as plsc

M = 204800
B = 4096
H = 256
SV = 65536
NC = 2
NS = 16
NW = NC * NS
C = 40
PER_W = M // NW
CHUNKS_PER_W = PER_W // C
N_CHUNKS = M // C

TK_BLK = 2048
TK_GRID = SV // TK_BLK


def _fold_img(img_z, role_emb, face_emb, w1i, b1):
    def body(img_ref, role_ref, f_ref, w1_ref, b1_ref, a_img, rf):
        a_img[...] = (jnp.dot(img_ref[...], w1_ref[...],
                              preferred_element_type=jnp.float32,
                              precision=lax.Precision.HIGHEST)
                      + b1_ref[...])
        role_rep = jnp.broadcast_to(role_ref[...][:, None, :], (8, 8, 16))
        role_rep = role_rep.reshape(64, 16)
        f_pad = jnp.concatenate(
            [f_ref[...], jnp.zeros((8, 8), jnp.float32)], axis=1)
        f_rep = jnp.broadcast_to(f_pad[None, :, :], (8, 8, 16))
        f_rep = f_rep.reshape(64, 16)
        rf[...] = jnp.concatenate(
            [role_rep, f_rep, jnp.zeros((64, 96), jnp.float32)], axis=1)

    full = lambda shape: pl.BlockSpec(shape, lambda: tuple(0 for _ in shape))
    return pl.pallas_call(
        body,
        in_specs=[full((B, 256)), full((8, 16)), full((8, 8)),
                  full((256, H)), full((1, H))],
        out_specs=[full((B, H)), full((64, 128))],
        out_shape=[
            jax.ShapeDtypeStruct((B, H), jnp.float32),
            jax.ShapeDtypeStruct((64, 128), jnp.float32),
        ],
    )(img_z, role_emb, face_emb, w1i, b1)


def _pad_tables(str_emb, t_emb, k_emb):
    def body(str_ref, t_ref, k_ref, str128, tk):
        z64 = jnp.zeros((TK_BLK, 64), jnp.float32)
        str128[...] = jnp.concatenate([str_ref[...], z64], axis=1)

        t_blk = t_ref[...]
        t_rep = jnp.broadcast_to(t_blk[:, None, :], (8, 256, 16))
        t_rep = t_rep.reshape(TK_BLK, 16)
        k_rep = jnp.broadcast_to(k_ref[...][None, :, :], (8, 256, 16))
        k_rep = k_rep.reshape(TK_BLK, 16)
        tk[...] = jnp.concatenate(
            [t_rep, k_rep, jnp.zeros((TK_BLK, 96), jnp.float32)], axis=1)

    return pl.pallas_call(
        body,
        grid=(TK_GRID,),
        in_specs=[
            pl.BlockSpec((TK_BLK, 64), lambda i: (i, 0)),
            pl.BlockSpec((8, 16), lambda i: (i, 0)),
            pl.BlockSpec((256, 16), lambda i: (0, 0)),
        ],
        out_specs=[
            pl.BlockSpec((TK_BLK, 128), lambda i: (i, 0)),
            pl.BlockSpec((TK_BLK, 128), lambda i: (i, 0)),
        ],
        out_shape=[
            jax.ShapeDtypeStruct((SV, 128), jnp.float32),
            jax.ShapeDtypeStruct((SV, 128), jnp.float32),
        ],
    )(str_emb, t_emb, k_emb)


def _sc_gather(idxs, a_img, rf_t, str_t, tk_t):
    mesh = plsc.VectorSubcoreMesh(core_axis_name="c", subcore_axis_name="s")

    fields = [(1, 256, 32), (2, 288, 64), (3, 352, 64), (4, 416, 64),
              (5, 480, 64), (6, 544, 32)]

    @functools.partial(
        pl.kernel,
        out_type=jax.ShapeDtypeStruct((M, 576), jnp.float32),
        mesh=mesh,
        scratch_types=[
            pltpu.VMEM((2, 8, C), jnp.int32),
            pltpu.VMEM((2, C, 576), jnp.float32),
            pltpu.VMEM((2, C, 128), jnp.float32),
            pltpu.VMEM((2, C, 128), jnp.float32),
            pltpu.VMEM((2, C, 128), jnp.float32),
            pltpu.VMEM((2, C, 128), jnp.float32),
            pltpu.VMEM((2, C, 128), jnp.float32),
            pltpu.VMEM((2, C, 128), jnp.float32),
            pltpu.SemaphoreType.DMA,
            pltpu.SemaphoreType.DMA,
            pltpu.SemaphoreType.DMA,
            pltpu.SemaphoreType.DMA,
        ],
    )
    def k(idxs_hbm, img_hbm, rf_hbm, str_hbm, tk_hbm, g_out,
          ibuf, pbuf, brf, bpred, bop, bnt, bpu, btk,
          gsem0, gsem1, wsem0, wsem1):
        wid = lax.axis_index("s") * NC + lax.axis_index("c")
        base = wid * CHUNKS_PER_W
        gsems = [gsem0, gsem1]
        wsems = [wsem0, wsem1]
        stage = [brf, bpred, bop, bnt, bpu, btk]
        tabs = [rf_hbm, str_hbm, str_hbm, str_hbm, str_hbm, tk_hbm]

        def fire(c, p):
            pltpu.sync_copy(idxs_hbm.at[base + c], ibuf.at[p])
            pltpu.async_copy(img_hbm.at[ibuf.at[p, 0]],
                             pbuf.at[p, :, pl.ds(0, 256)], gsems[p])
            for jj, (buf, tab) in enumerate(zip(stage, tabs)):
                pltpu.async_copy(tab.at[ibuf.at[p, jj + 1]], buf.at[p],
                                 gsems[p])

        def wait_gathers(p):
            pltpu.make_async_copy(img_hbm.at[pl.ds(0, C), :],
                                  pbuf.at[p, :, pl.ds(0, 256)],
                                  gsems[p]).wait()
            for buf, tab in zip(stage, tabs):
                pltpu.make_async_copy(str_hbm.at[pl.ds(0, C), :], buf.at[p],
                                      gsems[p]).wait()

        def pack(p):
            def row(r, carry2):
                for buf, (_, off, w) in zip(stage, fields):
                    for j in range(w // 16):
                        pbuf[p, r, pl.ds(off + j * 16, 16)] = (
                            buf[p, r, pl.ds(j * 16, 16)])
                return carry2
            lax.fori_loop(0, C, row, 0)

        def fire_write(c, p):
            pltpu.async_copy(pbuf.at[p],
                             g_out.at[pl.ds((base + c) * C, C), :], wsems[p])

        def wait_write(p):
            pltpu.make_async_copy(pbuf.at[p], g_out.at[pl.ds(0, C), :],
                                  wsems[p]).wait()

        fire(0, 0)
        fire(1, 1)
        H2 = CHUNKS_PER_W // 2

        def body(i2, carry):
            for p in (0, 1):
                c = i2 * 2 + p
                wait_gathers(p)
                pack(p)
                fire_write(c, p)

                @pl.when(i2 < H2 - 1)
                def _():
                    wait_write(p)
                    fire(c + 2, p)
            return carry

        lax.fori_loop(0, H2, body, 0)
        wait_write(0)
        wait_write(1)

    return k(idxs, a_img, rf_t, str_t, tk_t)


def _head(g_feat, w1rest, w2pad, b2):
    bm = 4096

    def body(g_ref, w1_ref, w2_ref, b2_ref, out_ref):
        x = g_ref[...]
        h = x[:, :256] + jnp.dot(x[:, 256:].astype(jnp.bfloat16),
                                 w1_ref[...].astype(jnp.bfloat16),
                                 preferred_element_type=jnp.float32)
        h = jnp.maximum(h, 0.0)
        mm = jnp.dot(h.astype(jnp.bfloat16), w2_ref[...],
                     preferred_element_type=jnp.float32)
        out_ref[...] = mm[:, 0:1] + b2_ref[...]

    out = pl.pallas_call(
        body,
        grid=(M // bm,),
        in_specs=[
            pl.BlockSpec((bm, 576), lambda i: (i, 0)),
            pl.BlockSpec((320, H), lambda i: (0, 0)),
            pl.BlockSpec((H, 128), lambda i: (0, 0)),
            pl.BlockSpec((1, 1), lambda i: (0, 0)),
        ],
        out_specs=pl.BlockSpec((bm, 1), lambda i: (i, 0)),
        out_shape=jax.ShapeDtypeStruct((M, 1), jnp.float32),
    )(g_feat, w1rest, w2pad, b2)
    return out[:, 0]


def kernel(img_z, desc_batch_idx, role_idx, pred_i, op_i, nt_i, pu_i,
           t_idx, k_idx, f_idx, role_emb, str_emb, t_emb, k_emb, face_emb,
           W1, b1, W2, b2):
    i32 = jnp.int32
    rf_i = role_idx.astype(i32) * 8 + f_idx.astype(i32)
    tk_i = t_idx.astype(i32) * 256 + k_idx.astype(i32)
    zeros = jnp.zeros((M,), i32)
    idxs = jnp.stack([
        desc_batch_idx.astype(i32), rf_i, pred_i.astype(i32),
        op_i.astype(i32), nt_i.astype(i32), pu_i.astype(i32), tk_i, zeros,
    ])
    idxs = idxs.reshape(8, N_CHUNKS, C).transpose(1, 0, 2)

    w1t = W1.T
    w1rest = jnp.concatenate([
        w1t[256:272],
        w1t[560:568],
        jnp.zeros((8, H), jnp.float32),
        w1t[272:528],
        w1t[528:560],
    ], axis=0)

    a_img, rf_t = _fold_img(img_z, role_emb, face_emb, w1t[0:256],
                            b1.reshape(1, H))
    str_t, tk_t = _pad_tables(str_emb, t_emb, k_emb)
    g = _sc_gather(idxs, a_img, rf_t, str_t, tk_t)
    w2pad = jnp.concatenate(
        [W2.reshape(H, 1), jnp.zeros((H, 127), jnp.float32)],
        axis=1).astype(jnp.bfloat16)
    return _head(g, w1rest, w2pad, b2.reshape(1, 1))


# speedup vs baseline: 1.7160x; 1.1456x over previous
import functools

import jax
import jax.numpy as jnp
from jax import lax
from jax.experimental import pallas as pl
from jax.experimental.pallas import tpu as pltpu
from jax.experimental.pallas import tpu_sc as plsc

M = 204800
B = 4096
H = 256
SV = 65536
NC = 2
NS = 16
NW = NC * NS
C = 40
PER_W = M // NW
CHUNKS_PER_W = PER_W // C
N_CHUNKS = M // C

TK_BLK = 2048
TK_GRID = SV // TK_BLK


def _fold_img(img_z, role_emb, face_emb, w1i, b1):
    def body(img_ref, role_ref, f_ref, w1_ref, b1_ref, a_img, rf):
        a = (jnp.dot(img_ref[...], w1_ref[...],
                     preferred_element_type=jnp.float32,
                     precision=lax.Precision.HIGHEST)
             + b1_ref[...])
        u_lo = lax.bitcast_convert_type(a[:, :128], jnp.int32)
        u_hi = lax.bitcast_convert_type(a[:, 128:], jnp.int32)

        def rnd(u):
            return u + jnp.int32(0x7FFF) + ((u >> 16) & 1)

        a_img[...] = (((rnd(u_lo) >> 16) & jnp.int32(0xFFFF))
                      | (rnd(u_hi) & jnp.int32(-65536)))
        role_rep = jnp.broadcast_to(role_ref[...][:, None, :], (8, 8, 16))
        role_rep = role_rep.reshape(64, 16)
        f_pad = jnp.concatenate(
            [f_ref[...], jnp.zeros((8, 8), jnp.float32)], axis=1)
        f_rep = jnp.broadcast_to(f_pad[None, :, :], (8, 8, 16))
        f_rep = f_rep.reshape(64, 16)
        rf[...] = jnp.concatenate(
            [role_rep, f_rep, jnp.zeros((64, 96), jnp.float32)], axis=1)

    full = lambda shape: pl.BlockSpec(shape, lambda: tuple(0 for _ in shape))
    return pl.pallas_call(
        body,
        in_specs=[full((B, 256)), full((8, 16)), full((8, 8)),
                  full((256, H)), full((1, H))],
        out_specs=[full((B, 128)), full((64, 128))],
        out_shape=[
            jax.ShapeDtypeStruct((B, 128), jnp.int32),
            jax.ShapeDtypeStruct((64, 128), jnp.float32),
        ],
    )(img_z, role_emb, face_emb, w1i, b1)


def _pad_tables(str_emb, t_emb, k_emb):
    def body(str_ref, t_ref, k_ref, str128, tk):
        z64 = jnp.zeros((TK_BLK, 64), jnp.float32)
        str128[...] = jnp.concatenate([str_ref[...], z64], axis=1)

        t_blk = t_ref[...]
        t_rep = jnp.broadcast_to(t_blk[:, None, :], (8, 256, 16))
        t_rep = t_rep.reshape(TK_BLK, 16)
        k_rep = jnp.broadcast_to(k_ref[...][None, :, :], (8, 256, 16))
        k_rep = k_rep.reshape(TK_BLK, 16)
        tk[...] = jnp.concatenate(
            [t_rep, k_rep, jnp.zeros((TK_BLK, 96), jnp.float32)], axis=1)

    return pl.pallas_call(
        body,
        grid=(TK_GRID,),
        in_specs=[
            pl.BlockSpec((TK_BLK, 64), lambda i: (i, 0)),
            pl.BlockSpec((8, 16), lambda i: (i, 0)),
            pl.BlockSpec((256, 16), lambda i: (0, 0)),
        ],
        out_specs=[
            pl.BlockSpec((TK_BLK, 128), lambda i: (i, 0)),
            pl.BlockSpec((TK_BLK, 128), lambda i: (i, 0)),
        ],
        out_shape=[
            jax.ShapeDtypeStruct((SV, 128), jnp.float32),
            jax.ShapeDtypeStruct((SV, 128), jnp.float32),
        ],
    )(str_emb, t_emb, k_emb)


def _sc_gather(i_img, i_rf, i_pred, i_op, i_nt, i_pu, i_tk,
               a_img, rf_t, str_t, tk_t):
    mesh = plsc.VectorSubcoreMesh(core_axis_name="c", subcore_axis_name="s")

    fields = [(0, 32), (32, 64), (96, 64), (160, 64), (224, 64), (288, 32)]

    @functools.partial(
        pl.kernel,
        out_type=[
            jax.ShapeDtypeStruct((M, 128), jnp.int32),
            jax.ShapeDtypeStruct((M, 320), jnp.float32),
        ],
        mesh=mesh,
        scratch_types=[
            pltpu.VMEM((2, C), jnp.int32),
            pltpu.VMEM((2, C), jnp.int32),
            pltpu.VMEM((2, C), jnp.int32),
            pltpu.VMEM((2, C), jnp.int32),
            pltpu.VMEM((2, C), jnp.int32),
            pltpu.VMEM((2, C), jnp.int32),
            pltpu.VMEM((2, C), jnp.int32),
            pltpu.VMEM((2, C, 128), jnp.int32),
            pltpu.VMEM((2, C, 320), jnp.float32),
            pltpu.VMEM((2, C, 128), jnp.float32),
            pltpu.VMEM((2, C, 128), jnp.float32),
            pltpu.VMEM((2, C, 128), jnp.float32),
            pltpu.VMEM((2, C, 128), jnp.float32),
            pltpu.VMEM((2, C, 128), jnp.float32),
            pltpu.VMEM((2, C, 128), jnp.float32),
            pltpu.SemaphoreType.DMA,
            pltpu.SemaphoreType.DMA,
            pltpu.SemaphoreType.DMA,
            pltpu.SemaphoreType.DMA,
            pltpu.SemaphoreType.DMA,
            pltpu.SemaphoreType.DMA,
        ],
    )
    def k(ix_img, ix_rf, ix_pred, ix_op, ix_nt, ix_pu, ix_tk,
          img_hbm, rf_hbm, str_hbm, tk_hbm, g_img, g_out,
          ib_img, ib_rf, ib_pred, ib_op, ib_nt, ib_pu, ib_tk,
          bimg, pbuf, brf, bpred, bop, bnt, bpu, btk,
          isem0, isem1, gsem0, gsem1, wsem0, wsem1):
        wid = lax.axis_index("s") * NC + lax.axis_index("c")
        base = wid * CHUNKS_PER_W
        isems = [isem0, isem1]
        gsems = [gsem0, gsem1]
        wsems = [wsem0, wsem1]
        ibufs = [ib_img, ib_rf, ib_pred, ib_op, ib_nt, ib_pu, ib_tk]
        ixs = [ix_img, ix_rf, ix_pred, ix_op, ix_nt, ix_pu, ix_tk]
        stage = [brf, bpred, bop, bnt, bpu, btk]
        tabs = [rf_hbm, str_hbm, str_hbm, str_hbm, str_hbm, tk_hbm]

        def fire(c, p):
            lo = pl.ds((base + c) * C, C)
            for ix, ib in zip(ixs, ibufs):
                pltpu.async_copy(ix.at[lo], ib.at[p], isems[p])
            for ix, ib in zip(ixs, ibufs):
                pltpu.make_async_copy(ix.at[lo], ib.at[p], isems[p]).wait()
            pltpu.async_copy(img_hbm.at[ib_img.at[p]], bimg.at[p], gsems[p])
            for buf, tab, ib in zip(stage, tabs, ibufs[1:]):
                pltpu.async_copy(tab.at[ib.at[p]], buf.at[p], gsems[p])

        def wait_gathers(p):
            pltpu.make_async_copy(img_hbm.at[pl.ds(0, C)], bimg.at[p],
                                  gsems[p]).wait()
            for buf in stage:
                pltpu.make_async_copy(str_hbm.at[pl.ds(0, C), :], buf.at[p],
                                      gsems[p]).wait()

        def pack(p):
            def row(r, carry2):
                for buf, (off, w) in zip(stage, fields):
                    for j in range(w // 16):
                        pbuf[p, r, pl.ds(off + j * 16, 16)] = (
                            buf[p, r, pl.ds(j * 16, 16)])
                return carry2
            lax.fori_loop(0, C, row, 0)

        def fire_write(c, p):
            lo = pl.ds((base + c) * C, C)
            pltpu.async_copy(bimg.at[p], g_img.at[lo, :], wsems[p])
            pltpu.async_copy(pbuf.at[p], g_out.at[lo, :], wsems[p])

        def wait_write(p):
            pltpu.make_async_copy(bimg.at[p], g_img.at[pl.ds(0, C), :],
                                  wsems[p]).wait()
            pltpu.make_async_copy(pbuf.at[p], g_out.at[pl.ds(0, C), :],
                                  wsems[p]).wait()

        fire(0, 0)
        fire(1, 1)
        H2 = CHUNKS_PER_W // 2

        def body(i2, carry):
            for p in (0, 1):
                c = i2 * 2 + p
                wait_gathers(p)
                pack(p)
                fire_write(c, p)

                @pl.when(i2 < H2 - 1)
                def _():
                    wait_write(p)
                    fire(c + 2, p)
            return carry

        lax.fori_loop(0, H2, body, 0)
        wait_write(0)
        wait_write(1)

    return k(i_img, i_rf, i_pred, i_op, i_nt, i_pu, i_tk,
             a_img, rf_t, str_t, tk_t)


def _head(g_feat, w1rest, w2pad, b2):
    bm = 4096
    g_img, g_rest = g_feat

    def body(gi_ref, gr_ref, w1_ref, w2_ref, b2_ref, out_ref):
        x = gi_ref[...]
        lo = lax.bitcast_convert_type(x << 16, jnp.float32)
        hi = lax.bitcast_convert_type(x & jnp.int32(-65536), jnp.float32)
        img = jnp.concatenate([lo, hi], axis=1)
        h = img + jnp.dot(gr_ref[...].astype(jnp.bfloat16),
                          w1_ref[...].astype(jnp.bfloat16),
                          preferred_element_type=jnp.float32)
        h = jnp.maximum(h, 0.0)
        mm = jnp.dot(h.astype(jnp.bfloat16), w2_ref[...],
                     preferred_element_type=jnp.float32)
        out_ref[...] = mm[:, 0:1] + b2_ref[...]

    out = pl.pallas_call(
        body,
        grid=(M // bm,),
        in_specs=[
            pl.BlockSpec((bm, 128), lambda i: (i, 0)),
            pl.BlockSpec((bm, 320), lambda i: (i, 0)),
            pl.BlockSpec((320, H), lambda i: (0, 0)),
            pl.BlockSpec((H, 128), lambda i: (0, 0)),
            pl.BlockSpec((1, 1), lambda i: (0, 0)),
        ],
        out_specs=pl.BlockSpec((bm, 1), lambda i: (i, 0)),
        out_shape=jax.ShapeDtypeStruct((M, 1), jnp.float32),
    )(g_img, g_rest, w1rest, w2pad, b2)
    return out[:, 0]


def kernel(img_z, desc_batch_idx, role_idx, pred_i, op_i, nt_i, pu_i,
           t_idx, k_idx, f_idx, role_emb, str_emb, t_emb, k_emb, face_emb,
           W1, b1, W2, b2):
    i32 = jnp.int32
    rf_i = role_idx.astype(i32) * 8 + f_idx.astype(i32)
    tk_i = t_idx.astype(i32) * 256 + k_idx.astype(i32)

    w1t = W1.T
    w1rest = jnp.concatenate([
        w1t[256:272],
        w1t[560:568],
        jnp.zeros((8, H), jnp.float32),
        w1t[272:528],
        w1t[528:560],
    ], axis=0)

    a_img, rf_t = _fold_img(img_z, role_emb, face_emb, w1t[0:256],
                            b1.reshape(1, H))
    str_t, tk_t = _pad_tables(str_emb, t_emb, k_emb)
    g = _sc_gather(desc_batch_idx.astype(i32), rf_i, pred_i.astype(i32),
                   op_i.astype(i32), nt_i.astype(i32), pu_i.astype(i32),
                   tk_i, a_img, rf_t, str_t, tk_t)
    w2pad = jnp.concatenate(
        [W2.reshape(H, 1), jnp.zeros((H, 127), jnp.float32)],
        axis=1).astype(jnp.bfloat16)
    return _head(g, w1rest, w2pad, b2.reshape(1, 1))


# speedup vs baseline: 1.7527x; 1.0214x over previous
import functools

import jax
import jax.numpy as jnp
from jax import lax
from jax.experimental import pallas as pl
from jax.experimental.pallas import tpu as pltpu
from jax.experimental.pallas import tpu_sc as plsc

M = 204800
B = 4096
H = 256
SV = 65536
NC = 2
NS = 16
NW = NC * NS
C = 40
PER_W = M // NW
CHUNKS_PER_W = PER_W // C
N_CHUNKS = M // C

TK_BLK = 2048
TK_GRID = SV // TK_BLK


def _fold_img(img_z, role_emb, face_emb, w1i, b1):
    def body(img_ref, role_ref, f_ref, w1_ref, b1_ref, a_img, rf):
        a = (jnp.dot(img_ref[...], w1_ref[...],
                     preferred_element_type=jnp.float32,
                     precision=lax.Precision.HIGHEST)
             + b1_ref[...])
        u_lo = lax.bitcast_convert_type(a[:, :128], jnp.int32)
        u_hi = lax.bitcast_convert_type(a[:, 128:], jnp.int32)

        def rnd(u):
            return u + jnp.int32(0x7FFF) + ((u >> 16) & 1)

        a_img[...] = (((rnd(u_lo) >> 16) & jnp.int32(0xFFFF))
                      | (rnd(u_hi) & jnp.int32(-65536)))
        role_rep = jnp.broadcast_to(role_ref[...][:, None, :], (8, 8, 16))
        role_rep = role_rep.reshape(64, 16)
        f_pad = jnp.concatenate(
            [f_ref[...], jnp.zeros((8, 8), jnp.float32)], axis=1)
        f_rep = jnp.broadcast_to(f_pad[None, :, :], (8, 8, 16))
        f_rep = f_rep.reshape(64, 16)
        rf[...] = jnp.concatenate(
            [role_rep, f_rep, jnp.zeros((64, 96), jnp.float32)], axis=1)

    full = lambda shape: pl.BlockSpec(shape, lambda: tuple(0 for _ in shape))
    return pl.pallas_call(
        body,
        in_specs=[full((B, 256)), full((8, 16)), full((8, 8)),
                  full((256, H)), full((1, H))],
        out_specs=[full((B, 128)), full((64, 128))],
        out_shape=[
            jax.ShapeDtypeStruct((B, 128), jnp.int32),
            jax.ShapeDtypeStruct((64, 128), jnp.float32),
        ],
    )(img_z, role_emb, face_emb, w1i, b1)


def _pad_tables(str_emb, t_emb, k_emb):
    def body(str_ref, t_ref, k_ref, str128, tk):
        z64 = jnp.zeros((TK_BLK, 64), jnp.float32)
        str128[...] = jnp.concatenate([str_ref[...], z64], axis=1)

        t_blk = t_ref[...]
        t_rep = jnp.broadcast_to(t_blk[:, None, :], (8, 256, 16))
        t_rep = t_rep.reshape(TK_BLK, 16)
        k_rep = jnp.broadcast_to(k_ref[...][None, :, :], (8, 256, 16))
        k_rep = k_rep.reshape(TK_BLK, 16)
        tk[...] = jnp.concatenate(
            [t_rep, k_rep, jnp.zeros((TK_BLK, 96), jnp.float32)], axis=1)

    return pl.pallas_call(
        body,
        grid=(TK_GRID,),
        in_specs=[
            pl.BlockSpec((TK_BLK, 64), lambda i: (i, 0)),
            pl.BlockSpec((8, 16), lambda i: (i, 0)),
            pl.BlockSpec((256, 16), lambda i: (0, 0)),
        ],
        out_specs=[
            pl.BlockSpec((TK_BLK, 128), lambda i: (i, 0)),
            pl.BlockSpec((TK_BLK, 128), lambda i: (i, 0)),
        ],
        out_shape=[
            jax.ShapeDtypeStruct((SV, 128), jnp.float32),
            jax.ShapeDtypeStruct((SV, 128), jnp.float32),
        ],
    )(str_emb, t_emb, k_emb)


def _sc_gather(i_img, i_rf, i_pred, i_op, i_nt, i_pu, i_tk,
               a_img, rf_t, str_t, tk_t):
    mesh = plsc.VectorSubcoreMesh(core_axis_name="c", subcore_axis_name="s")

    fields = [(0, 32), (32, 64), (96, 64), (160, 64), (224, 64), (288, 32)]

    @functools.partial(
        pl.kernel,
        out_type=[
            jax.ShapeDtypeStruct((M, 128), jnp.int32),
            jax.ShapeDtypeStruct((M, 160), jnp.int32),
        ],
        mesh=mesh,
        scratch_types=[
            pltpu.VMEM((2, C), jnp.int32),
            pltpu.VMEM((2, C), jnp.int32),
            pltpu.VMEM((2, C), jnp.int32),
            pltpu.VMEM((2, C), jnp.int32),
            pltpu.VMEM((2, C), jnp.int32),
            pltpu.VMEM((2, C), jnp.int32),
            pltpu.VMEM((2, C), jnp.int32),
            pltpu.VMEM((2, C, 128), jnp.int32),
            pltpu.VMEM((2, C, 160), jnp.int32),
            pltpu.VMEM((2, C, 128), jnp.float32),
            pltpu.VMEM((2, C, 128), jnp.float32),
            pltpu.VMEM((2, C, 128), jnp.float32),
            pltpu.VMEM((2, C, 128), jnp.float32),
            pltpu.VMEM((2, C, 128), jnp.float32),
            pltpu.VMEM((2, C, 128), jnp.float32),
            pltpu.SemaphoreType.DMA,
            pltpu.SemaphoreType.DMA,
            pltpu.SemaphoreType.DMA,
            pltpu.SemaphoreType.DMA,
            pltpu.SemaphoreType.DMA,
            pltpu.SemaphoreType.DMA,
        ],
    )
    def k(ix_img, ix_rf, ix_pred, ix_op, ix_nt, ix_pu, ix_tk,
          img_hbm, rf_hbm, str_hbm, tk_hbm, g_img, g_out,
          ib_img, ib_rf, ib_pred, ib_op, ib_nt, ib_pu, ib_tk,
          bimg, pbuf, brf, bpred, bop, bnt, bpu, btk,
          isem0, isem1, gsem0, gsem1, wsem0, wsem1):
        wid = lax.axis_index("s") * NC + lax.axis_index("c")
        base = wid * CHUNKS_PER_W
        isems = [isem0, isem1]
        gsems = [gsem0, gsem1]
        wsems = [wsem0, wsem1]
        ibufs = [ib_img, ib_rf, ib_pred, ib_op, ib_nt, ib_pu, ib_tk]
        ixs = [ix_img, ix_rf, ix_pred, ix_op, ix_nt, ix_pu, ix_tk]
        stage = [brf, bpred, bop, bnt, bpu, btk]
        tabs = [rf_hbm, str_hbm, str_hbm, str_hbm, str_hbm, tk_hbm]

        def fire(c, p):
            lo = pl.ds((base + c) * C, C)
            for ix, ib in zip(ixs, ibufs):
                pltpu.async_copy(ix.at[lo], ib.at[p], isems[p])
            for ix, ib in zip(ixs, ibufs):
                pltpu.make_async_copy(ix.at[lo], ib.at[p], isems[p]).wait()
            pltpu.async_copy(img_hbm.at[ib_img.at[p]], bimg.at[p], gsems[p])
            for buf, tab, ib in zip(stage, tabs, ibufs[1:]):
                pltpu.async_copy(tab.at[ib.at[p]], buf.at[p], gsems[p])

        def wait_gathers(p):
            pltpu.make_async_copy(img_hbm.at[pl.ds(0, C)], bimg.at[p],
                                  gsems[p]).wait()
            for buf in stage:
                pltpu.make_async_copy(str_hbm.at[pl.ds(0, C), :], buf.at[p],
                                      gsems[p]).wait()

        lo_src = [(brf, 0), (brf, 16), (bpred, 0), (bpred, 16), (bpred, 32),
                  (bpred, 48), (bop, 0), (bop, 16), (bop, 32), (bop, 48)]
        hi_src = [(bnt, 0), (bnt, 16), (bnt, 32), (bnt, 48), (bpu, 0),
                  (bpu, 16), (bpu, 32), (bpu, 48), (btk, 0), (btk, 16)]

        def rnd(u):
            return u + jnp.int32(0x7FFF) + ((u >> 16) & 1)

        def pack(p):
            def row(r, carry2):
                for j in range(10):
                    blo, clo = lo_src[j]
                    bhi, chi = hi_src[j]
                    vlo = lax.bitcast_convert_type(
                        blo[p, r, pl.ds(clo, 16)], jnp.int32)
                    vhi = lax.bitcast_convert_type(
                        bhi[p, r, pl.ds(chi, 16)], jnp.int32)
                    pbuf[p, r, pl.ds(j * 16, 16)] = (
                        ((rnd(vlo) >> 16) & jnp.int32(0xFFFF))
                        | (rnd(vhi) & jnp.int32(-65536)))
                return carry2
            lax.fori_loop(0, C, row, 0)

        def fire_write(c, p):
            lo = pl.ds((base + c) * C, C)
            pltpu.async_copy(bimg.at[p], g_img.at[lo, :], wsems[p])
            pltpu.async_copy(pbuf.at[p], g_out.at[lo, :], wsems[p])

        def wait_write(p):
            pltpu.make_async_copy(bimg.at[p], g_img.at[pl.ds(0, C), :],
                                  wsems[p]).wait()
            pltpu.make_async_copy(pbuf.at[p], g_out.at[pl.ds(0, C), :],
                                  wsems[p]).wait()

        fire(0, 0)
        fire(1, 1)
        H2 = CHUNKS_PER_W // 2

        def body(i2, carry):
            for p in (0, 1):
                c = i2 * 2 + p
                wait_gathers(p)
                pack(p)
                fire_write(c, p)

                @pl.when(i2 < H2 - 1)
                def _():
                    wait_write(p)
                    fire(c + 2, p)
            return carry

        lax.fori_loop(0, H2, body, 0)
        wait_write(0)
        wait_write(1)

    return k(i_img, i_rf, i_pred, i_op, i_nt, i_pu, i_tk,
             a_img, rf_t, str_t, tk_t)


def _head(g_feat, w1rest, w2pad, b2):
    bm = 4096
    g_img, g_rest = g_feat

    def body(gi_ref, gr_ref, w1_ref, w2_ref, b2_ref, out_ref):
        def unpack(x):
            lo = lax.bitcast_convert_type(x << 16, jnp.float32)
            hi = lax.bitcast_convert_type(x & jnp.int32(-65536), jnp.float32)
            return jnp.concatenate([lo, hi], axis=1)

        img = unpack(gi_ref[...])
        rest = unpack(gr_ref[...])
        h = img + jnp.dot(rest.astype(jnp.bfloat16),
                          w1_ref[...].astype(jnp.bfloat16),
                          preferred_element_type=jnp.float32)
        h = jnp.maximum(h, 0.0)
        mm = jnp.dot(h.astype(jnp.bfloat16), w2_ref[...],
                     preferred_element_type=jnp.float32)
        out_ref[...] = mm[:, 0:1] + b2_ref[...]

    out = pl.pallas_call(
        body,
        grid=(M // bm,),
        in_specs=[
            pl.BlockSpec((bm, 128), lambda i: (i, 0)),
            pl.BlockSpec((bm, 160), lambda i: (i, 0)),
            pl.BlockSpec((320, H), lambda i: (0, 0)),
            pl.BlockSpec((H, 128), lambda i: (0, 0)),
            pl.BlockSpec((1, 1), lambda i: (0, 0)),
        ],
        out_specs=pl.BlockSpec((bm, 1), lambda i: (i, 0)),
        out_shape=jax.ShapeDtypeStruct((M, 1), jnp.float32),
    )(g_img, g_rest, w1rest, w2pad, b2)
    return out[:, 0]


def kernel(img_z, desc_batch_idx, role_idx, pred_i, op_i, nt_i, pu_i,
           t_idx, k_idx, f_idx, role_emb, str_emb, t_emb, k_emb, face_emb,
           W1, b1, W2, b2):
    i32 = jnp.int32
    rf_i = role_idx.astype(i32) * 8 + f_idx.astype(i32)
    tk_i = t_idx.astype(i32) * 256 + k_idx.astype(i32)

    w1t = W1.T
    w1rest = jnp.concatenate([
        w1t[256:272],
        w1t[560:568],
        jnp.zeros((8, H), jnp.float32),
        w1t[272:528],
        w1t[528:560],
    ], axis=0)

    a_img, rf_t = _fold_img(img_z, role_emb, face_emb, w1t[0:256],
                            b1.reshape(1, H))
    str_t, tk_t = _pad_tables(str_emb, t_emb, k_emb)
    g = _sc_gather(desc_batch_idx.astype(i32), rf_i, pred_i.astype(i32),
                   op_i.astype(i32), nt_i.astype(i32), pu_i.astype(i32),
                   tk_i, a_img, rf_t, str_t, tk_t)
    w2pad = jnp.concatenate(
        [W2.reshape(H, 1), jnp.zeros((H, 127), jnp.float32)],
        axis=1).astype(jnp.bfloat16)
    return _head(g, w1rest, w2pad, b2.reshape(1, 1))


# speedup vs baseline: 1.8305x; 1.0444x over previous
import functools

import jax
import jax.numpy as jnp
from jax import lax
from jax.experimental import pallas as pl
from jax.experimental.pallas import tpu as pltpu
from jax.experimental.pallas import tpu_sc as plsc

M = 204800
B = 4096
H = 256
SV = 65536
NC = 2
NS = 16
NW = NC * NS
C = 40
PER_W = M // NW
CHUNKS_PER_W = PER_W // C
N_CHUNKS = M // C

TK_BLK = 2048
TK_GRID = SV // TK_BLK


def _fold_img(img_z, role_emb, face_emb, w1i, b1):
    def body(img_ref, role_ref, f_ref, w1_ref, b1_ref, a_img, rf):
        a = (jnp.dot(img_ref[...], w1_ref[...],
                     preferred_element_type=jnp.float32,
                     precision=lax.Precision.HIGHEST)
             + b1_ref[...])
        u_lo = lax.bitcast_convert_type(a[:, :128], jnp.int32)
        u_hi = lax.bitcast_convert_type(a[:, 128:], jnp.int32)

        def rnd(u):
            return u + jnp.int32(0x7FFF) + ((u >> 16) & 1)

        a_img[...] = (((rnd(u_lo) >> 16) & jnp.int32(0xFFFF))
                      | (rnd(u_hi) & jnp.int32(-65536)))
        role_rep = jnp.broadcast_to(role_ref[...][:, None, :], (8, 8, 16))
        role_rep = role_rep.reshape(64, 16)
        f_pad = jnp.concatenate(
            [f_ref[...], jnp.zeros((8, 8), jnp.float32)], axis=1)
        f_rep = jnp.broadcast_to(f_pad[None, :, :], (8, 8, 16))
        f_rep = f_rep.reshape(64, 16)
        rf[...] = jnp.concatenate(
            [role_rep, f_rep, jnp.zeros((64, 96), jnp.float32)], axis=1)

    full = lambda shape: pl.BlockSpec(shape, lambda: tuple(0 for _ in shape))
    return pl.pallas_call(
        body,
        in_specs=[full((B, 256)), full((8, 16)), full((8, 8)),
                  full((256, H)), full((1, H))],
        out_specs=[full((B, 128)), full((64, 128))],
        out_shape=[
            jax.ShapeDtypeStruct((B, 128), jnp.int32),
            jax.ShapeDtypeStruct((64, 128), jnp.float32),
        ],
    )(img_z, role_emb, face_emb, w1i, b1)


def _pad_tables(str_emb, t_emb, k_emb):
    def body(str_ref, t_ref, k_ref, str128, tk):
        z64 = jnp.zeros((TK_BLK, 64), jnp.float32)
        str128[...] = jnp.concatenate([str_ref[...], z64], axis=1)

        t_blk = t_ref[...]
        t_rep = jnp.broadcast_to(t_blk[:, None, :], (8, 256, 16))
        t_rep = t_rep.reshape(TK_BLK, 16)
        k_rep = jnp.broadcast_to(k_ref[...][None, :, :], (8, 256, 16))
        k_rep = k_rep.reshape(TK_BLK, 16)
        tk[...] = jnp.concatenate(
            [t_rep, k_rep, jnp.zeros((TK_BLK, 96), jnp.float32)], axis=1)

    return pl.pallas_call(
        body,
        grid=(TK_GRID,),
        in_specs=[
            pl.BlockSpec((TK_BLK, 64), lambda i: (i, 0)),
            pl.BlockSpec((8, 16), lambda i: (i, 0)),
            pl.BlockSpec((256, 16), lambda i: (0, 0)),
        ],
        out_specs=[
            pl.BlockSpec((TK_BLK, 128), lambda i: (i, 0)),
            pl.BlockSpec((TK_BLK, 128), lambda i: (i, 0)),
        ],
        out_shape=[
            jax.ShapeDtypeStruct((SV, 128), jnp.float32),
            jax.ShapeDtypeStruct((SV, 128), jnp.float32),
        ],
    )(str_emb, t_emb, k_emb)


def _sc_gather(i_img, i_rf, i_pred, i_op, i_nt, i_pu, i_tk,
               a_img, rf_t, str_t, tk_t, m):
    mesh = plsc.VectorSubcoreMesh(core_axis_name="c", subcore_axis_name="s")
    chunks_per_w = m // (NW * C)

    fields = [(0, 32), (32, 64), (96, 64), (160, 64), (224, 64), (288, 32)]

    @functools.partial(
        pl.kernel,
        out_type=[
            jax.ShapeDtypeStruct((m, 128), jnp.int32),
            jax.ShapeDtypeStruct((m, 160), jnp.int32),
        ],
        mesh=mesh,
        scratch_types=[
            pltpu.VMEM((2, C), jnp.int32),
            pltpu.VMEM((2, C), jnp.int32),
            pltpu.VMEM((2, C), jnp.int32),
            pltpu.VMEM((2, C), jnp.int32),
            pltpu.VMEM((2, C), jnp.int32),
            pltpu.VMEM((2, C), jnp.int32),
            pltpu.VMEM((2, C), jnp.int32),
            pltpu.VMEM((2, C, 128), jnp.int32),
            pltpu.VMEM((2, C, 160), jnp.int32),
            pltpu.VMEM((2, C, 128), jnp.float32),
            pltpu.VMEM((2, C, 128), jnp.float32),
            pltpu.VMEM((2, C, 128), jnp.float32),
            pltpu.VMEM((2, C, 128), jnp.float32),
            pltpu.VMEM((2, C, 128), jnp.float32),
            pltpu.VMEM((2, C, 128), jnp.float32),
            pltpu.SemaphoreType.DMA,
            pltpu.SemaphoreType.DMA,
            pltpu.SemaphoreType.DMA,
            pltpu.SemaphoreType.DMA,
            pltpu.SemaphoreType.DMA,
            pltpu.SemaphoreType.DMA,
        ],
    )
    def k(ix_img, ix_rf, ix_pred, ix_op, ix_nt, ix_pu, ix_tk,
          img_hbm, rf_hbm, str_hbm, tk_hbm, g_img, g_out,
          ib_img, ib_rf, ib_pred, ib_op, ib_nt, ib_pu, ib_tk,
          bimg, pbuf, brf, bpred, bop, bnt, bpu, btk,
          isem0, isem1, gsem0, gsem1, wsem0, wsem1):
        wid = lax.axis_index("s") * NC + lax.axis_index("c")
        base = wid * chunks_per_w
        isems = [isem0, isem1]
        gsems = [gsem0, gsem1]
        wsems = [wsem0, wsem1]
        ibufs = [ib_img, ib_rf, ib_pred, ib_op, ib_nt, ib_pu, ib_tk]
        ixs = [ix_img, ix_rf, ix_pred, ix_op, ix_nt, ix_pu, ix_tk]
        stage = [brf, bpred, bop, bnt, bpu, btk]
        tabs = [rf_hbm, str_hbm, str_hbm, str_hbm, str_hbm, tk_hbm]

        def fire(c, p):
            lo = pl.ds((base + c) * C, C)
            for ix, ib in zip(ixs, ibufs):
                pltpu.async_copy(ix.at[lo], ib.at[p], isems[p])
            for ix, ib in zip(ixs, ibufs):
                pltpu.make_async_copy(ix.at[lo], ib.at[p], isems[p]).wait()
            pltpu.async_copy(img_hbm.at[ib_img.at[p]], bimg.at[p], gsems[p])
            for buf, tab, ib in zip(stage, tabs, ibufs[1:]):
                pltpu.async_copy(tab.at[ib.at[p]], buf.at[p], gsems[p])

        def wait_gathers(p):
            pltpu.make_async_copy(img_hbm.at[pl.ds(0, C)], bimg.at[p],
                                  gsems[p]).wait()
            for buf in stage:
                pltpu.make_async_copy(str_hbm.at[pl.ds(0, C), :], buf.at[p],
                                      gsems[p]).wait()

        lo_src = [(brf, 0), (brf, 16), (bpred, 0), (bpred, 16), (bpred, 32),
                  (bpred, 48), (bop, 0), (bop, 16), (bop, 32), (bop, 48)]
        hi_src = [(bnt, 0), (bnt, 16), (bnt, 32), (bnt, 48), (bpu, 0),
                  (bpu, 16), (bpu, 32), (bpu, 48), (btk, 0), (btk, 16)]

        def rnd(u):
            return u + jnp.int32(0x7FFF) + ((u >> 16) & 1)

        def pack(p):
            def row(r, carry2):
                for j in range(10):
                    blo, clo = lo_src[j]
                    bhi, chi = hi_src[j]
                    vlo = lax.bitcast_convert_type(
                        blo[p, r, pl.ds(clo, 16)], jnp.int32)
                    vhi = lax.bitcast_convert_type(
                        bhi[p, r, pl.ds(chi, 16)], jnp.int32)
                    pbuf[p, r, pl.ds(j * 16, 16)] = (
                        ((rnd(vlo) >> 16) & jnp.int32(0xFFFF))
                        | (rnd(vhi) & jnp.int32(-65536)))
                return carry2
            lax.fori_loop(0, C, row, 0)

        def fire_write(c, p):
            lo = pl.ds((base + c) * C, C)
            pltpu.async_copy(bimg.at[p], g_img.at[lo, :], wsems[p])
            pltpu.async_copy(pbuf.at[p], g_out.at[lo, :], wsems[p])

        def wait_write(p):
            pltpu.make_async_copy(bimg.at[p], g_img.at[pl.ds(0, C), :],
                                  wsems[p]).wait()
            pltpu.make_async_copy(pbuf.at[p], g_out.at[pl.ds(0, C), :],
                                  wsems[p]).wait()

        fire(0, 0)
        fire(1, 1)
        H2 = chunks_per_w // 2

        def body(i2, carry):
            for p in (0, 1):
                c = i2 * 2 + p
                wait_gathers(p)
                pack(p)
                fire_write(c, p)

                @pl.when(i2 < H2 - 1)
                def _():
                    wait_write(p)
                    fire(c + 2, p)
            return carry

        lax.fori_loop(0, H2, body, 0)
        wait_write(0)
        wait_write(1)

    return k(i_img, i_rf, i_pred, i_op, i_nt, i_pu, i_tk,
             a_img, rf_t, str_t, tk_t)


def _head(g_feat, w1rest, w2pad, b2, m):
    bm = 4096
    g_img, g_rest = g_feat

    def body(gi_ref, gr_ref, w1_ref, w2_ref, b2_ref, out_ref):
        def unpack(x):
            lo = lax.bitcast_convert_type(x << 16, jnp.float32)
            hi = lax.bitcast_convert_type(x & jnp.int32(-65536), jnp.float32)
            return jnp.concatenate([lo, hi], axis=1)

        img = unpack(gi_ref[...])
        rest = unpack(gr_ref[...])
        h = img + jnp.dot(rest.astype(jnp.bfloat16),
                          w1_ref[...].astype(jnp.bfloat16),
                          preferred_element_type=jnp.float32)
        h = jnp.maximum(h, 0.0)
        mm = jnp.dot(h.astype(jnp.bfloat16), w2_ref[...],
                     preferred_element_type=jnp.float32)
        out_ref[...] = mm[:, 0:1] + b2_ref[...]

    out = pl.pallas_call(
        body,
        grid=(m // bm,),
        in_specs=[
            pl.BlockSpec((bm, 128), lambda i: (i, 0)),
            pl.BlockSpec((bm, 160), lambda i: (i, 0)),
            pl.BlockSpec((320, H), lambda i: (0, 0)),
            pl.BlockSpec((H, 128), lambda i: (0, 0)),
            pl.BlockSpec((1, 1), lambda i: (0, 0)),
        ],
        out_specs=pl.BlockSpec((bm, 1), lambda i: (i, 0)),
        out_shape=jax.ShapeDtypeStruct((m, 1), jnp.float32),
    )(g_img, g_rest, w1rest, w2pad, b2)
    return out[:, 0]


def kernel(img_z, desc_batch_idx, role_idx, pred_i, op_i, nt_i, pu_i,
           t_idx, k_idx, f_idx, role_emb, str_emb, t_emb, k_emb, face_emb,
           W1, b1, W2, b2):
    i32 = jnp.int32
    rf_i = role_idx.astype(i32) * 8 + f_idx.astype(i32)
    tk_i = t_idx.astype(i32) * 256 + k_idx.astype(i32)

    w1t = W1.T
    w1rest = jnp.concatenate([
        w1t[256:272],
        w1t[560:568],
        jnp.zeros((8, H), jnp.float32),
        w1t[272:528],
        w1t[528:560],
    ], axis=0)

    a_img, rf_t = _fold_img(img_z, role_emb, face_emb, w1t[0:256],
                            b1.reshape(1, H))
    str_t, tk_t = _pad_tables(str_emb, t_emb, k_emb)
    w2pad = jnp.concatenate(
        [W2.reshape(H, 1), jnp.zeros((H, 127), jnp.float32)],
        axis=1).astype(jnp.bfloat16)

    idx = [desc_batch_idx.astype(i32), rf_i, pred_i.astype(i32),
           op_i.astype(i32), nt_i.astype(i32), pu_i.astype(i32), tk_i]
    MH = M // 2
    outs = []
    for h_ in range(2):
        sl = slice(h_ * MH, (h_ + 1) * MH)
        g = _sc_gather(*[a[sl] for a in idx], a_img, rf_t, str_t, tk_t, MH)
        outs.append(_head(g, w1rest, w2pad, b2.reshape(1, 1), MH))
    return jnp.concatenate(outs)


# speedup vs baseline: 1.8336x; 1.0017x over previous
import functools

import jax
import jax.numpy as jnp
from jax import lax
from jax.experimental import pallas as pl
from jax.experimental.pallas import tpu as pltpu
from jax.experimental.pallas import tpu_sc as plsc

M = 204800
B = 4096
H = 256
SV = 65536
NC = 2
NS = 16
NW = NC * NS
C = 40
PER_W = M // NW
CHUNKS_PER_W = PER_W // C
N_CHUNKS = M // C

TK_BLK = 2048
TK_GRID = SV // TK_BLK


def _fold_img(img_z, role_emb, face_emb, w1i, b1):
    def body(img_ref, role_ref, f_ref, w1_ref, b1_ref, a_img, rf):
        a = (jnp.dot(img_ref[...], w1_ref[...],
                     preferred_element_type=jnp.float32,
                     precision=lax.Precision.HIGHEST)
             + b1_ref[...])
        u_lo = lax.bitcast_convert_type(a[:, :128], jnp.int32)
        u_hi = lax.bitcast_convert_type(a[:, 128:], jnp.int32)

        def rnd(u):
            return u + jnp.int32(0x7FFF) + ((u >> 16) & 1)

        a_img[...] = (((rnd(u_lo) >> 16) & jnp.int32(0xFFFF))
                      | (rnd(u_hi) & jnp.int32(-65536)))
        role_rep = jnp.broadcast_to(role_ref[...][:, None, :], (8, 8, 16))
        role_rep = role_rep.reshape(64, 16)
        f_pad = jnp.concatenate(
            [f_ref[...], jnp.zeros((8, 8), jnp.float32)], axis=1)
        f_rep = jnp.broadcast_to(f_pad[None, :, :], (8, 8, 16))
        f_rep = f_rep.reshape(64, 16)
        rf[...] = jnp.concatenate(
            [role_rep, f_rep, jnp.zeros((64, 96), jnp.float32)], axis=1)

    full = lambda shape: pl.BlockSpec(shape, lambda: tuple(0 for _ in shape))
    return pl.pallas_call(
        body,
        in_specs=[full((B, 256)), full((8, 16)), full((8, 8)),
                  full((256, H)), full((1, H))],
        out_specs=[full((B, 128)), full((64, 128))],
        out_shape=[
            jax.ShapeDtypeStruct((B, 128), jnp.int32),
            jax.ShapeDtypeStruct((64, 128), jnp.float32),
        ],
    )(img_z, role_emb, face_emb, w1i, b1)


def _pad_tables(str_emb, t_emb, k_emb):
    def body(str_ref, t_ref, k_ref, str128, tk):
        z64 = jnp.zeros((TK_BLK, 64), jnp.float32)
        str128[...] = jnp.concatenate([str_ref[...], z64], axis=1)

        t_blk = t_ref[...]
        t_rep = jnp.broadcast_to(t_blk[:, None, :], (8, 256, 16))
        t_rep = t_rep.reshape(TK_BLK, 16)
        k_rep = jnp.broadcast_to(k_ref[...][None, :, :], (8, 256, 16))
        k_rep = k_rep.reshape(TK_BLK, 16)
        tk[...] = jnp.concatenate(
            [t_rep, k_rep, jnp.zeros((TK_BLK, 96), jnp.float32)], axis=1)

    return pl.pallas_call(
        body,
        grid=(TK_GRID,),
        in_specs=[
            pl.BlockSpec((TK_BLK, 64), lambda i: (i, 0)),
            pl.BlockSpec((8, 16), lambda i: (i, 0)),
            pl.BlockSpec((256, 16), lambda i: (0, 0)),
        ],
        out_specs=[
            pl.BlockSpec((TK_BLK, 128), lambda i: (i, 0)),
            pl.BlockSpec((TK_BLK, 128), lambda i: (i, 0)),
        ],
        out_shape=[
            jax.ShapeDtypeStruct((SV, 128), jnp.float32),
            jax.ShapeDtypeStruct((SV, 128), jnp.float32),
        ],
    )(str_emb, t_emb, k_emb)


def _sc_gather(i_img, i_rf, i_pred, i_op, i_nt, i_pu, i_tk,
               a_img, rf_t, str_t, tk_t, m):
    mesh = plsc.VectorSubcoreMesh(core_axis_name="c", subcore_axis_name="s")
    chunks_per_w = m // (NW * C)

    fields = [(0, 32), (32, 64), (96, 64), (160, 64), (224, 64), (288, 32)]

    @functools.partial(
        pl.kernel,
        out_type=[
            jax.ShapeDtypeStruct((m, 128), jnp.int32),
            jax.ShapeDtypeStruct((m, 160), jnp.int32),
        ],
        mesh=mesh,
        scratch_types=[
            pltpu.VMEM((2, C), jnp.int32),
            pltpu.VMEM((2, C), jnp.int32),
            pltpu.VMEM((2, C), jnp.int32),
            pltpu.VMEM((2, C), jnp.int32),
            pltpu.VMEM((2, C), jnp.int32),
            pltpu.VMEM((2, C), jnp.int32),
            pltpu.VMEM((2, C), jnp.int32),
            pltpu.VMEM((2, C, 128), jnp.int32),
            pltpu.VMEM((2, C, 160), jnp.int32),
            pltpu.VMEM((2, C, 128), jnp.float32),
            pltpu.VMEM((2, C, 128), jnp.float32),
            pltpu.VMEM((2, C, 128), jnp.float32),
            pltpu.VMEM((2, C, 128), jnp.float32),
            pltpu.VMEM((2, C, 128), jnp.float32),
            pltpu.VMEM((2, C, 128), jnp.float32),
            pltpu.SemaphoreType.DMA,
            pltpu.SemaphoreType.DMA,
            pltpu.SemaphoreType.DMA,
            pltpu.SemaphoreType.DMA,
            pltpu.SemaphoreType.DMA,
            pltpu.SemaphoreType.DMA,
        ],
    )
    def k(ix_img, ix_rf, ix_pred, ix_op, ix_nt, ix_pu, ix_tk,
          img_hbm, rf_hbm, str_hbm, tk_hbm, g_img, g_out,
          ib_img, ib_rf, ib_pred, ib_op, ib_nt, ib_pu, ib_tk,
          bimg, pbuf, brf, bpred, bop, bnt, bpu, btk,
          isem0, isem1, gsem0, gsem1, wsem0, wsem1):
        wid = lax.axis_index("s") * NC + lax.axis_index("c")
        base = wid * chunks_per_w
        isems = [isem0, isem1]
        gsems = [gsem0, gsem1]
        wsems = [wsem0, wsem1]
        ibufs = [ib_img, ib_rf, ib_pred, ib_op, ib_nt, ib_pu, ib_tk]
        ixs = [ix_img, ix_rf, ix_pred, ix_op, ix_nt, ix_pu, ix_tk]
        stage = [brf, bpred, bop, bnt, bpu, btk]
        tabs = [rf_hbm, str_hbm, str_hbm, str_hbm, str_hbm, tk_hbm]

        def fire(c, p):
            lo = pl.ds((base + c) * C, C)
            for ix, ib in zip(ixs, ibufs):
                pltpu.async_copy(ix.at[lo], ib.at[p], isems[p])
            for ix, ib in zip(ixs, ibufs):
                pltpu.make_async_copy(ix.at[lo], ib.at[p], isems[p]).wait()
            pltpu.async_copy(img_hbm.at[ib_img.at[p]], bimg.at[p], gsems[p])
            for buf, tab, ib in zip(stage, tabs, ibufs[1:]):
                pltpu.async_copy(tab.at[ib.at[p]], buf.at[p], gsems[p])

        def wait_gathers(p):
            pltpu.make_async_copy(img_hbm.at[pl.ds(0, C)], bimg.at[p],
                                  gsems[p]).wait()
            for buf in stage:
                pltpu.make_async_copy(str_hbm.at[pl.ds(0, C), :], buf.at[p],
                                      gsems[p]).wait()

        lo_src = [(brf, 0), (brf, 16), (bpred, 0), (bpred, 16), (bpred, 32),
                  (bpred, 48), (bop, 0), (bop, 16), (bop, 32), (bop, 48)]
        hi_src = [(bnt, 0), (bnt, 16), (bnt, 32), (bnt, 48), (bpu, 0),
                  (bpu, 16), (bpu, 32), (bpu, 48), (btk, 0), (btk, 16)]

        def rnd(u):
            return u + jnp.int32(0x7FFF) + ((u >> 16) & 1)

        def pack(p):
            def row(r, carry2):
                for j in range(10):
                    blo, clo = lo_src[j]
                    bhi, chi = hi_src[j]
                    vlo = lax.bitcast_convert_type(
                        blo[p, r, pl.ds(clo, 16)], jnp.int32)
                    vhi = lax.bitcast_convert_type(
                        bhi[p, r, pl.ds(chi, 16)], jnp.int32)
                    pbuf[p, r, pl.ds(j * 16, 16)] = (
                        ((rnd(vlo) >> 16) & jnp.int32(0xFFFF))
                        | (rnd(vhi) & jnp.int32(-65536)))
                return carry2
            lax.fori_loop(0, C, row, 0)

        def fire_write(c, p):
            lo = pl.ds((base + c) * C, C)
            pltpu.async_copy(bimg.at[p], g_img.at[lo, :], wsems[p])
            pltpu.async_copy(pbuf.at[p], g_out.at[lo, :], wsems[p])

        def wait_write(p):
            pltpu.make_async_copy(bimg.at[p], g_img.at[pl.ds(0, C), :],
                                  wsems[p]).wait()
            pltpu.make_async_copy(pbuf.at[p], g_out.at[pl.ds(0, C), :],
                                  wsems[p]).wait()

        fire(0, 0)
        fire(1, 1)
        H2 = chunks_per_w // 2

        def body(i2, carry):
            for p in (0, 1):
                c = i2 * 2 + p
                wait_gathers(p)
                pack(p)
                fire_write(c, p)

                @pl.when(i2 < H2 - 1)
                def _():
                    wait_write(p)
                    fire(c + 2, p)
            return carry

        lax.fori_loop(0, H2, body, 0)
        wait_write(0)
        wait_write(1)

    return k(i_img, i_rf, i_pred, i_op, i_nt, i_pu, i_tk,
             a_img, rf_t, str_t, tk_t)


def _head(g_feat, w1rest, w2pad, b2, m):
    bm = 4096
    g_img, g_rest = g_feat

    def body(gi_ref, gr_ref, w1_ref, w2_ref, b2_ref, out_ref):
        def unpack(x):
            lo = lax.bitcast_convert_type(x << 16, jnp.float32)
            hi = lax.bitcast_convert_type(x & jnp.int32(-65536), jnp.float32)
            return jnp.concatenate([lo, hi], axis=1)

        img = unpack(gi_ref[...])
        rest = unpack(gr_ref[...])
        h = img + jnp.dot(rest.astype(jnp.bfloat16),
                          w1_ref[...].astype(jnp.bfloat16),
                          preferred_element_type=jnp.float32)
        h = jnp.maximum(h, 0.0)
        mm = jnp.dot(h.astype(jnp.bfloat16), w2_ref[...],
                     preferred_element_type=jnp.float32)
        out_ref[...] = mm[:, 0:1] + b2_ref[...]

    out = pl.pallas_call(
        body,
        grid=(m // bm,),
        in_specs=[
            pl.BlockSpec((bm, 128), lambda i: (i, 0)),
            pl.BlockSpec((bm, 160), lambda i: (i, 0)),
            pl.BlockSpec((320, H), lambda i: (0, 0)),
            pl.BlockSpec((H, 128), lambda i: (0, 0)),
            pl.BlockSpec((1, 1), lambda i: (0, 0)),
        ],
        out_specs=pl.BlockSpec((bm, 1), lambda i: (i, 0)),
        out_shape=jax.ShapeDtypeStruct((m, 1), jnp.float32),
    )(g_img, g_rest, w1rest, w2pad, b2)
    return out[:, 0]


def kernel(img_z, desc_batch_idx, role_idx, pred_i, op_i, nt_i, pu_i,
           t_idx, k_idx, f_idx, role_emb, str_emb, t_emb, k_emb, face_emb,
           W1, b1, W2, b2):
    i32 = jnp.int32
    rf_i = role_idx.astype(i32) * 8 + f_idx.astype(i32)
    tk_i = t_idx.astype(i32) * 256 + k_idx.astype(i32)

    w1t = W1.T
    w1rest = jnp.concatenate([
        w1t[256:272],
        w1t[560:568],
        jnp.zeros((8, H), jnp.float32),
        w1t[272:528],
        w1t[528:560],
    ], axis=0)

    a_img, rf_t = _fold_img(img_z, role_emb, face_emb, w1t[0:256],
                            b1.reshape(1, H))
    str_t, tk_t = _pad_tables(str_emb, t_emb, k_emb)
    w2pad = jnp.concatenate(
        [W2.reshape(H, 1), jnp.zeros((H, 127), jnp.float32)],
        axis=1).astype(jnp.bfloat16)

    idx = [desc_batch_idx.astype(i32), rf_i, pred_i.astype(i32),
           op_i.astype(i32), nt_i.astype(i32), pu_i.astype(i32), tk_i]
    MH = M // 4
    outs = []
    for h_ in range(4):
        sl = slice(h_ * MH, (h_ + 1) * MH)
        g = _sc_gather(*[a[sl] for a in idx], a_img, rf_t, str_t, tk_t, MH)
        outs.append(_head(g, w1rest, w2pad, b2.reshape(1, 1), MH))
    return jnp.concatenate(outs)
